# baseline (speedup 1.0000x reference)
import functools

import jax
import jax.numpy as jnp
from jax import lax
from jax.experimental import pallas as pl
from jax.experimental.pallas import tpu as pltpu
from jax.experimental.pallas import tpu_sc as plsc

_N = 10000
_E = 320000
_D_IN = 128
_D_H = 128
_D_CLS = 64

_NC = 2
_NS = 16
_K = 80
_EPC = _E // _NC
_EPT = _EPC // _NS
_ITERS = _EPT // _K
_NCHUNK = _N // _K

_CW = 16

_mesh = plsc.VectorSubcoreMesh(core_axis_name="c", subcore_axis_name="s")


def _fill_vmem(ref, rows, width, value):
    per_row = width // 16

    def body(i, _):
        r = i // per_row
        col = (i % per_row) * 16
        ref[r, pl.ds(col, 16)] = jnp.full((16,), value, jnp.float32)
        return 0

    lax.fori_loop(0, rows * per_row, body, 0)


def _fill_vmem1d(ref, n, value):
    def body(i, _):
        ref[pl.ds(i * 16, 16)] = jnp.full((16,), value, jnp.float32)
        return 0

    lax.fori_loop(0, n // 16, body, 0)


def _for_my_chunks(s, fn):
    def body(i, _):
        t = i * _NS + s

        @pl.when(t < _NCHUNK)
        def _():
            fn(t * _K)

        return 0

    lax.fori_loop(0, (_NCHUNK + _NS - 1) // _NS, body, 0)


_NPAD = 10240
_NB = _NPAD // (8 * 128)
_DK = 125
_DI = 80
_DB = 16


def _sc_degrees_body(src_hbm, dst_hbm, out_hbm,
                     idx_s, idx_d, ones, stage, acc_s, acc_d, sem_a, sem_b):
    c = lax.axis_index("c")
    s = lax.axis_index("s")
    w = c * _NS + s

    _fill_vmem(stage, 8, 128, 0.0)
    nz = _NPAD // 128

    def zero_chunk(i, _):
        t = i * _NS + s

        @pl.when(t < nz)
        def _():
            off = pl.multiple_of(t * 128, 128)
            pltpu.sync_copy(stage.at[0], acc_s.at[pl.ds(off, 128)])
            pltpu.sync_copy(stage.at[0], acc_d.at[pl.ds(off, 128)])

        return 0

    lax.fori_loop(0, (nz + _NS - 1) // _NS, zero_chunk, 0)
    _fill_vmem1d(ones, 128, 1.0)
    pay = ones.at[pl.ds(0, _DK)]
    plsc.subcore_barrier()

    def blk(bi, _):
        b = pl.multiple_of(bi * _DB, 8)
        pltpu.sync_copy(src_hbm.at[w, pl.ds(b, _DB)], idx_s)
        pltpu.sync_copy(dst_hbm.at[w, pl.ds(b, _DB)], idx_d)

        def step(j, _):
            pltpu.async_copy(pay, acc_s.at[idx_s.at[j]], sem_a, add=True)
            pltpu.async_copy(pay, acc_d.at[idx_d.at[j]], sem_b, add=True)

            @pl.when(j >= 2)
            def _():
                pltpu.make_async_copy(pay, acc_s.at[idx_s.at[j - 2]], sem_a).wait()
                pltpu.make_async_copy(pay, acc_d.at[idx_d.at[j - 2]], sem_b).wait()

            return 0

        lax.fori_loop(0, _DB, step, 0)
        for j in (_DB - 2, _DB - 1):
            pltpu.make_async_copy(pay, acc_s.at[idx_s.at[j]], sem_a).wait()
            pltpu.make_async_copy(pay, acc_d.at[idx_d.at[j]], sem_b).wait()
        return 0

    lax.fori_loop(0, _DI // _DB, blk, 0)
    plsc.subcore_barrier()

    def out_block(acc, sect, blk):
        for r in range(8):
            off = pl.multiple_of(blk * 1024 + r * 128, 128)
            pltpu.sync_copy(acc.at[pl.ds(off, 128)], stage.at[r])
        row0 = pl.multiple_of(sect * (8 * _NB) + blk * 8, 8)
        pltpu.sync_copy(stage, out_hbm.at[pl.ds(row0, 8)])

    def out_step(i, _):
        t = i * _NS + s

        @pl.when(t < _NB)
        def _():
            out_block(acc_s, c, t)

        @pl.when((t >= _NB) & (t < 2 * _NB))
        def _():
            out_block(acc_d, 2 + c, t - _NB)

        return 0

    lax.fori_loop(0, (2 * _NB + _NS - 1) // _NS, out_step, 0)


_sc_degrees = pl.kernel(
    _sc_degrees_body,
    out_type=jax.ShapeDtypeStruct((4 * 8 * _NB, 128), jnp.float32),
    mesh=_mesh,
    scratch_types=[
        pltpu.VMEM((_DB, _DK), jnp.int32),
        pltpu.VMEM((_DB, _DK), jnp.int32),
        pltpu.VMEM((128,), jnp.float32),
        pltpu.VMEM((8, 128), jnp.float32),
        pltpu.VMEM_SHARED((_NPAD,), jnp.float32),
        pltpu.VMEM_SHARED((_NPAD,), jnp.float32),
        pltpu.SemaphoreType.DMA,
        pltpu.SemaphoreType.DMA,
    ],
)


_GK = 50
_GI = 200
_GB = 8
_ZK = 40


def _sc_agg_body(t_hbm, src_hbm, dst_hbm, out_hbm,
                 idx_s, idx_d, r0, r1, r2, r3, acc,
                 g0, g1, g2, g3, s0, s1, s2, s3, *, d):
    c = lax.axis_index("c")
    s = lax.axis_index("s")
    w = c * _NS + s
    bufs = (r0, r1, r2, r3)
    gsems = (g0, g1, g2, g3)
    ssems = (s0, s1, s2, s3)

    _fill_vmem(r0, _ZK, d, 0.0)
    nz = _N // _ZK

    def zero_chunk(i, _):
        t = i * _NS + s

        @pl.when(t < nz)
        def _():
            b = pl.multiple_of(t * _ZK, 8)
            pltpu.sync_copy(r0.at[pl.ds(0, _ZK)], acc.at[pl.ds(b, _ZK)])

        return 0

    lax.fori_loop(0, (nz + _NS - 1) // _NS, zero_chunk, 0)
    plsc.subcore_barrier()

    def gstart(j, rows, sem):
        pltpu.async_copy(t_hbm.at[idx_s.at[j]], rows, sem)

    def gwait(j, rows, sem):
        pltpu.make_async_copy(t_hbm.at[idx_s.at[j]], rows, sem).wait()

    def sstart(j, rows, sem):
        pltpu.async_copy(rows, acc.at[idx_d.at[j]], sem, add=True)

    def swait(j, rows, sem):
        pltpu.make_async_copy(rows, acc.at[idx_d.at[j]], sem).wait()

    def blkfn(bi, _):
        b = pl.multiple_of(bi * _GB, 8)
        pltpu.sync_copy(src_hbm.at[w, pl.ds(b, _GB)], idx_s)
        pltpu.sync_copy(dst_hbm.at[w, pl.ds(b, _GB)], idx_d)
        gstart(0, bufs[0], gsems[0])
        gstart(1, bufs[1], gsems[1])
        for j in range(_GB):
            m = j % 4
            gwait(j, bufs[m], gsems[m])
            sstart(j, bufs[m], ssems[m])
            if j + 2 < _GB:
                if j >= 2:
                    p = (j - 2) % 4
                    swait(j - 2, bufs[p], ssems[p])
                n = (j + 2) % 4
                gstart(j + 2, bufs[n], gsems[n])
        for j in range(_GB - 4, _GB):
            m = j % 4
            swait(j, bufs[m], ssems[m])
        return 0

    lax.fori_loop(0, _GI // _GB, blkfn, 0)
    plsc.subcore_barrier()

    _for_my_chunks(
        s,
        lambda base: pltpu.sync_copy(
            acc.at[pl.ds(base, _K)], out_hbm.at[c, pl.ds(base, _K)]
        ),
    )


def _make_sc_agg(d):
    return pl.kernel(
        functools.partial(_sc_agg_body, d=d),
        out_type=jax.ShapeDtypeStruct((_NC, _N, d), jnp.float32),
        mesh=_mesh,
        scratch_types=[
            pltpu.VMEM((_GB, _GK), jnp.int32),
            pltpu.VMEM((_GB, _GK), jnp.int32),
            pltpu.VMEM((_GK, d), jnp.float32),
            pltpu.VMEM((_GK, d), jnp.float32),
            pltpu.VMEM((_GK, d), jnp.float32),
            pltpu.VMEM((_GK, d), jnp.float32),
            pltpu.VMEM_SHARED((_N, d), jnp.float32),
            pltpu.SemaphoreType.DMA,
            pltpu.SemaphoreType.DMA,
            pltpu.SemaphoreType.DMA,
            pltpu.SemaphoreType.DMA,
            pltpu.SemaphoreType.DMA,
            pltpu.SemaphoreType.DMA,
            pltpu.SemaphoreType.DMA,
            pltpu.SemaphoreType.DMA,
        ],
    )


_sc_agg128 = _make_sc_agg(_D_H)


def _scale_cols(cnt_ref, kind):
    k = 2 * kind
    c = cnt_ref[:, k : k + 1] + cnt_ref[:, k + 1 : k + 2]
    return lax.rsqrt(jnp.maximum(c, 1.0))


def _tc_first_body(x_ref, w_ref, cnt_ref, o_ref):
    so = _scale_cols(cnt_ref, 0)
    o_ref[...] = (
        jnp.dot(x_ref[...], w_ref[...], preferred_element_type=jnp.float32) * so
    )


def _tc_mid_body(p_ref, cnt_ref, b_ref, w_ref, o_ref):
    si = _scale_cols(cnt_ref, 1)
    so = _scale_cols(cnt_ref, 0)
    h = jnp.maximum((p_ref[0] + p_ref[1]) * si + b_ref[...], 0.0)
    o_ref[...] = (
        jnp.dot(h, w_ref[...], preferred_element_type=jnp.float32) * so
    )


def _tc_final_body(p_ref, cnt_ref, b_ref, o_ref):
    si = _scale_cols(cnt_ref, 1)
    logits = (p_ref[0] + p_ref[1])[:, : _D_CLS] * si + b_ref[...]
    m = jnp.max(logits, axis=-1, keepdims=True)
    lg = logits - m
    o_ref[...] = lg - jnp.log(jnp.sum(jnp.exp(lg), axis=-1, keepdims=True))


def _tc_first(x, w, cnt):
    return pl.pallas_call(
        _tc_first_body,
        out_shape=jax.ShapeDtypeStruct((_N, w.shape[1]), jnp.float32),
    )(x, w, cnt)


def _tc_mid(p, cnt, b, w):
    return pl.pallas_call(
        _tc_mid_body,
        out_shape=jax.ShapeDtypeStruct((_N, w.shape[1]), jnp.float32),
    )(p, cnt, b, w)


def _tc_final(p, cnt, b):
    return pl.pallas_call(
        _tc_final_body,
        out_shape=jax.ShapeDtypeStruct((_N, _D_CLS), jnp.float32),
    )(p, cnt, b)


def kernel(x, edge_index, W1, b1, W2, b2, W3, b3):
    nw = _NC * _NS
    src = edge_index[0].astype(jnp.int32).reshape(nw, _DI, _DK)
    dst = edge_index[1].astype(jnp.int32).reshape(nw, _DI, _DK)
    srca = edge_index[0].astype(jnp.int32).reshape(nw, _GI, _GK)
    dsta = edge_index[1].astype(jnp.int32).reshape(nw, _GI, _GK)
    x = x.astype(jnp.float32)
    b1r = b1.reshape(1, _D_H)
    b2r = b2.reshape(1, _D_H)
    b3r = b3.reshape(1, _D_CLS)
    W3p = jnp.pad(W3, ((0, 0), (0, _D_H - _D_CLS)))

    cnt = _sc_degrees(src, dst)
    cnt = cnt.reshape(4, _NPAD)[:, :_N].T
    t1 = _tc_first(x, W1, cnt)
    p1 = _sc_agg128(t1, srca, dsta)
    t2 = _tc_mid(p1, cnt, b1r, W2)
    p2 = _sc_agg128(t2, srca, dsta)
    t3 = _tc_mid(p2, cnt, b2r, W3p)
    p3 = _sc_agg128(t3, srca, dsta)
    return _tc_final(p3, cnt, b3r)


# speedup vs baseline: 1.1782x; 1.1782x over previous
import functools

import jax
import jax.numpy as jnp
from jax import lax
from jax.experimental import pallas as pl
from jax.experimental.pallas import tpu as pltpu
from jax.experimental.pallas import tpu_sc as plsc

_N = 10000
_E = 320000
_D_IN = 128
_D_H = 128
_D_CLS = 64

_NC = 2
_NS = 16
_K = 80
_EPC = _E // _NC
_EPT = _EPC // _NS
_ITERS = _EPT // _K
_NCHUNK = _N // _K

_CW = 16

_mesh = plsc.VectorSubcoreMesh(core_axis_name="c", subcore_axis_name="s")


def _fill_vmem(ref, rows, width, value):
    per_row = width // 16

    def body(i, _):
        r = i // per_row
        col = (i % per_row) * 16
        ref[r, pl.ds(col, 16)] = jnp.full((16,), value, jnp.float32)
        return 0

    lax.fori_loop(0, rows * per_row, body, 0)


def _fill_vmem1d(ref, n, value):
    def body(i, _):
        ref[pl.ds(i * 16, 16)] = jnp.full((16,), value, jnp.float32)
        return 0

    lax.fori_loop(0, n // 16, body, 0)


def _for_my_chunks(s, fn):
    def body(i, _):
        t = i * _NS + s

        @pl.when(t < _NCHUNK)
        def _():
            fn(t * _K)

        return 0

    lax.fori_loop(0, (_NCHUNK + _NS - 1) // _NS, body, 0)


_NPAD = 10240
_NB = _NPAD // (8 * 128)
_DK = 125
_DI = 80
_DB = 16


def _sc_degrees_body(src_hbm, dst_hbm, out_hbm,
                     idx_s, idx_d, ones, stage, acc_s, acc_d, sem_a, sem_b):
    c = lax.axis_index("c")
    s = lax.axis_index("s")
    w = c * _NS + s

    _fill_vmem(stage, 8, 128, 0.0)
    nz = _NPAD // 128

    def zero_chunk(i, _):
        t = i * _NS + s

        @pl.when(t < nz)
        def _():
            off = pl.multiple_of(t * 128, 128)
            pltpu.sync_copy(stage.at[0], acc_s.at[pl.ds(off, 128)])
            pltpu.sync_copy(stage.at[0], acc_d.at[pl.ds(off, 128)])

        return 0

    lax.fori_loop(0, (nz + _NS - 1) // _NS, zero_chunk, 0)
    _fill_vmem1d(ones, 128, 1.0)
    pay = ones.at[pl.ds(0, _DK)]
    plsc.subcore_barrier()

    def blk(bi, _):
        b = pl.multiple_of(bi * _DB, 8)
        pltpu.sync_copy(src_hbm.at[w, pl.ds(b, _DB)], idx_s)
        pltpu.sync_copy(dst_hbm.at[w, pl.ds(b, _DB)], idx_d)

        def step(j, _):
            pltpu.async_copy(pay, acc_s.at[idx_s.at[j]], sem_a, add=True)
            pltpu.async_copy(pay, acc_d.at[idx_d.at[j]], sem_b, add=True)

            @pl.when(j >= 2)
            def _():
                pltpu.make_async_copy(pay, acc_s.at[idx_s.at[j - 2]], sem_a).wait()
                pltpu.make_async_copy(pay, acc_d.at[idx_d.at[j - 2]], sem_b).wait()

            return 0

        lax.fori_loop(0, _DB, step, 0)
        for j in (_DB - 2, _DB - 1):
            pltpu.make_async_copy(pay, acc_s.at[idx_s.at[j]], sem_a).wait()
            pltpu.make_async_copy(pay, acc_d.at[idx_d.at[j]], sem_b).wait()
        return 0

    lax.fori_loop(0, _DI // _DB, blk, 0)
    plsc.subcore_barrier()

    def out_block(acc, sect, blk):
        for r in range(8):
            off = pl.multiple_of(blk * 1024 + r * 128, 128)
            pltpu.sync_copy(acc.at[pl.ds(off, 128)], stage.at[r])
        row0 = pl.multiple_of(sect * (8 * _NB) + blk * 8, 8)
        pltpu.sync_copy(stage, out_hbm.at[pl.ds(row0, 8)])

    def out_step(i, _):
        t = i * _NS + s

        @pl.when(t < _NB)
        def _():
            out_block(acc_s, c, t)

        @pl.when((t >= _NB) & (t < 2 * _NB))
        def _():
            out_block(acc_d, 2 + c, t - _NB)

        return 0

    lax.fori_loop(0, (2 * _NB + _NS - 1) // _NS, out_step, 0)


_sc_degrees = pl.kernel(
    _sc_degrees_body,
    out_type=jax.ShapeDtypeStruct((4 * 8 * _NB, 128), jnp.float32),
    mesh=_mesh,
    scratch_types=[
        pltpu.VMEM((_DB, _DK), jnp.int32),
        pltpu.VMEM((_DB, _DK), jnp.int32),
        pltpu.VMEM((128,), jnp.float32),
        pltpu.VMEM((8, 128), jnp.float32),
        pltpu.VMEM_SHARED((_NPAD,), jnp.float32),
        pltpu.VMEM_SHARED((_NPAD,), jnp.float32),
        pltpu.SemaphoreType.DMA,
        pltpu.SemaphoreType.DMA,
    ],
)


_GK = 125
_GI = 80
_GB = 16
_ZK = 40


def _sc_agg_body(t_hbm, src_hbm, dst_hbm, out_hbm,
                 idx_s, idx_d, r0, r1, acc,
                 g0, g1, s0, s1, *, d):
    c = lax.axis_index("c")
    s = lax.axis_index("s")
    w = c * _NS + s
    bufs = (r0, r1)
    gsems = (g0, g1)
    ssems = (s0, s1)

    _fill_vmem(r0, _ZK, d, 0.0)
    nz = _N // _ZK

    def zero_chunk(i, _):
        t = i * _NS + s

        @pl.when(t < nz)
        def _():
            b = pl.multiple_of(t * _ZK, 8)
            pltpu.sync_copy(r0.at[pl.ds(0, _ZK)], acc.at[pl.ds(b, _ZK)])

        return 0

    lax.fori_loop(0, (nz + _NS - 1) // _NS, zero_chunk, 0)
    plsc.subcore_barrier()

    def gstart(j, rows, sem):
        pltpu.async_copy(t_hbm.at[idx_s.at[j]], rows, sem)

    def gwait(j, rows, sem):
        pltpu.make_async_copy(t_hbm.at[idx_s.at[j]], rows, sem).wait()

    def sstart(j, rows, sem):
        pltpu.async_copy(rows, acc.at[idx_d.at[j]], sem, add=True)

    def swait(j, rows, sem):
        pltpu.make_async_copy(rows, acc.at[idx_d.at[j]], sem).wait()

    def blkfn(bi, _):
        b = pl.multiple_of(bi * _GB, 8)
        pltpu.sync_copy(src_hbm.at[w, pl.ds(b, _GB)], idx_s)
        pltpu.sync_copy(dst_hbm.at[w, pl.ds(b, _GB)], idx_d)
        gstart(0, bufs[0], gsems[0])
        for j in range(_GB):
            m = j % 2
            gwait(j, bufs[m], gsems[m])
            sstart(j, bufs[m], ssems[m])
            if j + 1 < _GB:
                n = (j + 1) % 2
                if j >= 1:
                    swait(j - 1, bufs[n], ssems[n])
                gstart(j + 1, bufs[n], gsems[n])
        for j in (_GB - 2, _GB - 1):
            m = j % 2
            swait(j, bufs[m], ssems[m])
        return 0

    lax.fori_loop(0, _GI // _GB, blkfn, 0)
    plsc.subcore_barrier()

    _for_my_chunks(
        s,
        lambda base: pltpu.sync_copy(
            acc.at[pl.ds(base, _K)], out_hbm.at[c, pl.ds(base, _K)]
        ),
    )


def _make_sc_agg(d):
    return pl.kernel(
        functools.partial(_sc_agg_body, d=d),
        out_type=jax.ShapeDtypeStruct((_NC, _N, d), jnp.float32),
        mesh=_mesh,
        scratch_types=[
            pltpu.VMEM((_GB, _GK), jnp.int32),
            pltpu.VMEM((_GB, _GK), jnp.int32),
            pltpu.VMEM((_GK, d), jnp.float32),
            pltpu.VMEM((_GK, d), jnp.float32),
            pltpu.VMEM_SHARED((_N, d), jnp.float32),
            pltpu.SemaphoreType.DMA,
            pltpu.SemaphoreType.DMA,
            pltpu.SemaphoreType.DMA,
            pltpu.SemaphoreType.DMA,
        ],
    )


_sc_agg128 = _make_sc_agg(_D_H)


def _scale_cols(cnt_ref, kind):
    k = 2 * kind
    c = cnt_ref[:, k : k + 1] + cnt_ref[:, k + 1 : k + 2]
    return lax.rsqrt(jnp.maximum(c, 1.0))


def _tc_first_body(x_ref, w_ref, cnt_ref, o_ref):
    so = _scale_cols(cnt_ref, 0)
    o_ref[...] = (
        jnp.dot(x_ref[...], w_ref[...], preferred_element_type=jnp.float32) * so
    )


def _tc_mid_body(p_ref, cnt_ref, b_ref, w_ref, o_ref):
    si = _scale_cols(cnt_ref, 1)
    so = _scale_cols(cnt_ref, 0)
    h = jnp.maximum((p_ref[0] + p_ref[1]) * si + b_ref[...], 0.0)
    o_ref[...] = (
        jnp.dot(h, w_ref[...], preferred_element_type=jnp.float32) * so
    )


def _tc_final_body(p_ref, cnt_ref, b_ref, o_ref):
    si = _scale_cols(cnt_ref, 1)
    logits = (p_ref[0] + p_ref[1])[:, : _D_CLS] * si + b_ref[...]
    m = jnp.max(logits, axis=-1, keepdims=True)
    lg = logits - m
    o_ref[...] = lg - jnp.log(jnp.sum(jnp.exp(lg), axis=-1, keepdims=True))


def _tc_first(x, w, cnt):
    return pl.pallas_call(
        _tc_first_body,
        out_shape=jax.ShapeDtypeStruct((_N, w.shape[1]), jnp.float32),
    )(x, w, cnt)


def _tc_mid(p, cnt, b, w):
    return pl.pallas_call(
        _tc_mid_body,
        out_shape=jax.ShapeDtypeStruct((_N, w.shape[1]), jnp.float32),
    )(p, cnt, b, w)


def _tc_final(p, cnt, b):
    return pl.pallas_call(
        _tc_final_body,
        out_shape=jax.ShapeDtypeStruct((_N, _D_CLS), jnp.float32),
    )(p, cnt, b)


def kernel(x, edge_index, W1, b1, W2, b2, W3, b3):
    nw = _NC * _NS
    src = edge_index[0].astype(jnp.int32).reshape(nw, _DI, _DK)
    dst = edge_index[1].astype(jnp.int32).reshape(nw, _DI, _DK)
    srca = edge_index[0].astype(jnp.int32).reshape(nw, _GI, _GK)
    dsta = edge_index[1].astype(jnp.int32).reshape(nw, _GI, _GK)
    x = x.astype(jnp.float32)
    b1r = b1.reshape(1, _D_H)
    b2r = b2.reshape(1, _D_H)
    b3r = b3.reshape(1, _D_CLS)
    W3p = jnp.pad(W3, ((0, 0), (0, _D_H - _D_CLS)))

    cnt = _sc_degrees(src, dst)
    cnt = cnt.reshape(4, _NPAD)[:, :_N].T
    t1 = _tc_first(x, W1, cnt)
    p1 = _sc_agg128(t1, srca, dsta)
    t2 = _tc_mid(p1, cnt, b1r, W2)
    p2 = _sc_agg128(t2, srca, dsta)
    t3 = _tc_mid(p2, cnt, b2r, W3p)
    p3 = _sc_agg128(t3, srca, dsta)
    return _tc_final(p3, cnt, b3r)
